# baseline (speedup 1.0000x reference)
import functools

import jax
import jax.numpy as jnp
from jax import lax
from jax.experimental import pallas as pl
from jax.experimental.pallas import tpu as pltpu
from jax.experimental.pallas import tpu_sc as plsc

_NC = 2
_NS = 16
_NW = _NC * _NS
_CHUNK = 112


def _sc_gather(tables, idx_r, n_pad, k, w):
    mesh = plsc.VectorSubcoreMesh(
        core_axis_name="c", subcore_axis_name="s",
        num_cores=_NC, num_subcores=_NS)
    dt = tables[0].dtype
    nt = len(tables)
    ns = 2 * nt
    out_sds = jax.ShapeDtypeStruct((n_pad, w), dt)

    @functools.partial(
        pl.kernel,
        mesh=mesh,
        out_type=[out_sds] * ns,
        scratch_types=(
            [pltpu.VMEM((k, _CHUNK), jnp.int32)] * 2
            + [pltpu.VMEM((_CHUNK, w), dt)] * (2 * ns)
            + [pltpu.SemaphoreType.DMA] * 4
        ),
    )
    def gather_kernel(*refs):
        tabs_in = refs[:nt]
        idx_hbm = refs[nt]
        outs = refs[nt + 1:nt + 1 + ns]
        i0_v, i1_v = refs[nt + 1 + ns:nt + 3 + ns]
        bufs = refs[nt + 3 + ns:nt + 3 + ns + 2 * ns]
        bufs_a, bufs_b = bufs[:ns], bufs[ns:]
        gsa, gsb, ssa, ssb = refs[-4:]

        wid = lax.axis_index("s") * _NC + lax.axis_index("c")
        pltpu.sync_copy(idx_hbm.at[0, wid], i0_v)
        pltpu.sync_copy(idx_hbm.at[1, wid], i1_v)
        tabs = tuple(t for t in tabs_in for _ in range(2))
        idxs = (i0_v, i1_v) * nt
        base0 = wid * (k * _CHUNK)

        def gissue(j, bufs, sem):
            for t in range(ns):
                pltpu.async_copy(tabs[t].at[idxs[t].at[j]], bufs[t], sem)

        def gwait(bufs, sem):
            for t in range(ns):
                pltpu.make_async_copy(
                    tabs[t].at[idxs[t].at[0]], bufs[t], sem).wait()

        def sissue(j, bufs, sem):
            for t in range(ns):
                pltpu.async_copy(
                    bufs[t], outs[t].at[pl.ds(base0 + j * _CHUNK, _CHUNK)],
                    sem)

        def sdrain(bufs, sem):
            for t in range(ns):
                pltpu.make_async_copy(
                    bufs[t], outs[t].at[pl.ds(0, _CHUNK)], sem).wait()

        gissue(0, bufs_a, gsa)

        def body(it, carry):
            ja = 2 * it
            jb = ja + 1
            jn = lax.min(ja + 2, k - 1)
            gwait(bufs_a, gsa)

            @pl.when(it > 0)
            def _():
                sdrain(bufs_b, ssb)

            gissue(jb, bufs_b, gsb)
            sissue(ja, bufs_a, ssa)
            gwait(bufs_b, gsb)
            sdrain(bufs_a, ssa)
            gissue(jn, bufs_a, gsa)
            sissue(jb, bufs_b, ssb)
            return carry

        lax.fori_loop(0, k // 2, body, 0)
        gwait(bufs_a, gsa)
        sdrain(bufs_b, ssb)

    return gather_kernel(*tables, idx_r)


def _tc_fused(x, gathered, w_iou, ui0, ui1, uf0, uf1, ufb, biou,
              prev_out, off_blocks, slice_blocks, n, hs, block):
    grid = (slice_blocks,)
    packed = len(gathered) == 2
    f32 = jnp.float32

    def gates(x_ref, w_refs, out_ref, h0b, h1b, c0v, c1v):
        wiou_ref, ui0_ref, ui1_ref, uf0_ref, uf1_ref, ufb_ref, biou_ref = \
            w_refs
        fpre = (jnp.dot(h0b, uf0_ref[...], preferred_element_type=f32)
                + jnp.dot(h1b, uf1_ref[...], preferred_element_type=f32)
                + ufb_ref[...])
        f0 = jax.nn.sigmoid(fpre[:, :hs])
        f1 = jax.nn.sigmoid(fpre[:, hs:])
        cred = f0 * c0v + f1 * c1v
        iou = (jnp.dot(x_ref[...], wiou_ref[...], preferred_element_type=f32)
               + jnp.dot(h0b, ui0_ref[...], preferred_element_type=f32)
               + jnp.dot(h1b, ui1_ref[...], preferred_element_type=f32)
               + biou_ref[...])
        i = jax.nn.sigmoid(iou[:, :hs])
        o = jax.nn.sigmoid(iou[:, hs:2 * hs])
        u = jnp.tanh(iou[:, 2 * hs:])
        c = i * u + cred
        h = o * jnp.tanh(c)
        out_ref[:, :hs] = h
        out_ref[:, hs:] = c

    if packed:
        def body(x_ref, hc0_ref, hc1_ref, *rest):
            hc0b = hc0_ref[...]
            hc1b = hc1_ref[...]
            gates(x_ref, rest[:7], rest[-1],
                  hc0b[:, :hs], hc1b[:, :hs],
                  hc0b[:, hs:].astype(f32), hc1b[:, hs:].astype(f32))
        g_specs = [pl.BlockSpec((block, 2 * hs), lambda i: (i, 0))] * 2
    else:
        def body(x_ref, h0_ref, h1_ref, c0_ref, c1_ref, *rest):
            gates(x_ref, rest[:7], rest[-1],
                  h0_ref[...], h1_ref[...],
                  c0_ref[...].astype(f32), c1_ref[...].astype(f32))
        g_specs = [pl.BlockSpec((block, hs), lambda i: (i, 0))] * 4

    row_g = lambda i: (i + off_blocks, 0)
    full = lambda i: (0, 0)
    in_specs = (
        [pl.BlockSpec((block, x.shape[1]), row_g)]
        + g_specs
        + [pl.BlockSpec(w.shape, full)
           for w in (w_iou, ui0, ui1, uf0, uf1, ufb, biou)]
    )
    args = [x, *gathered, w_iou, ui0, ui1, uf0, uf1, ufb, biou]
    aliases = {}
    if prev_out is not None:
        in_specs.append(pl.BlockSpec(memory_space=pl.ANY))
        aliases = {len(args): 0}
        args.append(prev_out)
    return pl.pallas_call(
        body,
        grid=grid,
        in_specs=in_specs,
        out_specs=pl.BlockSpec((block, 2 * hs), row_g),
        out_shape=jax.ShapeDtypeStruct((n, 2 * hs), jnp.float32),
        input_output_aliases=aliases,
        compiler_params=pltpu.CompilerParams(
            dimension_semantics=("arbitrary",)),
    )(*args)


_BLOCK = 2048
_KS = 8


def kernel(x, h_all, c_all, child_idx, W_iou, U_iou, U_f_w, U_f_b, b_iou):
    n, _ = x.shape
    hs = h_all.shape[1]
    bf = jnp.bfloat16
    f32 = jnp.float32

    k = pl.cdiv(n, _NW * _CHUNK)
    k += k & 1
    n_pad = _NW * k * _CHUNK

    idx32 = child_idx.astype(jnp.int32)
    idx_t = jnp.pad(idx32.T, ((0, 0), (0, n_pad - n)))

    ks_list = []
    rem = k
    while rem > 0:
        ks_list.append(min(_KS, rem))
        rem -= ks_list[-1]

    ui0, ui1 = U_iou[:hs], U_iou[hs:]
    uf0, uf1 = U_f_w[:hs], U_f_w[hs:]
    ufb = U_f_b.reshape(1, 2 * hs)

    to_i32 = lambda a: jax.lax.bitcast_convert_type(
        a.astype(bf).reshape(n, hs // 2, 2), jnp.int32)
    hc16 = jnp.concatenate([to_i32(h_all), to_i32(c_all)], axis=1)
    un_i32 = lambda a, rows: jax.lax.bitcast_convert_type(
        a, bf).reshape(rows, 2 * hs)
    ui0_16, ui1_16 = ui0.astype(bf), ui1.astype(bf)
    uf0_16, uf1_16 = uf0.astype(bf), uf1.astype(bf)

    out = None
    base = 0
    for s, ks in enumerate(ks_list):
        rows = _NW * ks * _CHUNK
        idx_r = idx_t[:, base:base + rows].reshape(2, _NW, ks, _CHUNK)
        if s == 0:
            gathered = _sc_gather([h_all, c_all], idx_r, rows, ks, hs)
            ws = (ui0, ui1, uf0, uf1)
        else:
            hc0, hc1 = _sc_gather([hc16], idx_r, rows, ks, hs)
            gathered = (un_i32(hc0, rows), un_i32(hc1, rows))
            ws = (ui0_16, ui1_16, uf0_16, uf1_16)
        out = _tc_fused(x, gathered, W_iou, ws[0], ws[1], ws[2], ws[3],
                        ufb, b_iou, out, base // _BLOCK, rows // _BLOCK,
                        n, hs, _BLOCK)
        base += rows
    return out


# speedup vs baseline: 5.4121x; 5.4121x over previous
import functools

import jax
import jax.numpy as jnp
from jax import lax
from jax.experimental import pallas as pl
from jax.experimental.pallas import tpu as pltpu
from jax.experimental.pallas import tpu_sc as plsc

_NC = 2
_NS = 16
_NW = _NC * _NS
_CHUNK = 112


def _sc_gather(tables, idx_r, n_pad, k, w):
    mesh = plsc.VectorSubcoreMesh(
        core_axis_name="c", subcore_axis_name="s",
        num_cores=_NC, num_subcores=_NS)
    dt = tables[0].dtype
    nt = len(tables)
    ns = 2 * nt
    out_sds = jax.ShapeDtypeStruct((n_pad, w), dt)

    @functools.partial(
        pl.kernel,
        mesh=mesh,
        out_type=[out_sds] * ns,
        scratch_types=(
            [pltpu.VMEM((k, _CHUNK), jnp.int32)] * 2
            + [pltpu.VMEM((_CHUNK, w), dt)] * (2 * ns)
            + [pltpu.SemaphoreType.DMA] * 4
        ),
    )
    def gather_kernel(*refs):
        tabs_in = refs[:nt]
        idx_hbm = refs[nt]
        outs = refs[nt + 1:nt + 1 + ns]
        i0_v, i1_v = refs[nt + 1 + ns:nt + 3 + ns]
        bufs = refs[nt + 3 + ns:nt + 3 + ns + 2 * ns]
        bufs_a, bufs_b = bufs[:ns], bufs[ns:]
        gsa, gsb, ssa, ssb = refs[-4:]

        wid = lax.axis_index("s") * _NC + lax.axis_index("c")
        pltpu.sync_copy(idx_hbm.at[0, wid], i0_v)
        pltpu.sync_copy(idx_hbm.at[1, wid], i1_v)
        tabs = tuple(t for t in tabs_in for _ in range(2))
        idxs = (i0_v, i1_v) * nt
        base0 = wid * (k * _CHUNK)

        def gissue(j, bufs, sem):
            for t in range(ns):
                pltpu.async_copy(tabs[t].at[idxs[t].at[j]], bufs[t], sem)

        def gwait(bufs, sem):
            for t in range(ns):
                pltpu.make_async_copy(
                    tabs[t].at[idxs[t].at[0]], bufs[t], sem).wait()

        def sissue(j, bufs, sem):
            for t in range(ns):
                pltpu.async_copy(
                    bufs[t], outs[t].at[pl.ds(base0 + j * _CHUNK, _CHUNK)],
                    sem)

        def sdrain(bufs, sem):
            for t in range(ns):
                pltpu.make_async_copy(
                    bufs[t], outs[t].at[pl.ds(0, _CHUNK)], sem).wait()

        gissue(0, bufs_a, gsa)

        def body(it, carry):
            ja = 2 * it
            jb = ja + 1
            jn = lax.min(ja + 2, k - 1)
            gwait(bufs_a, gsa)

            @pl.when(it > 0)
            def _():
                sdrain(bufs_b, ssb)

            gissue(jb, bufs_b, gsb)
            sissue(ja, bufs_a, ssa)
            gwait(bufs_b, gsb)
            sdrain(bufs_a, ssa)
            gissue(jn, bufs_a, gsa)
            sissue(jb, bufs_b, ssb)
            return carry

        lax.fori_loop(0, k // 2, body, 0)
        gwait(bufs_a, gsa)
        sdrain(bufs_b, ssb)

    return gather_kernel(*tables, idx_r)


def _tc_pack(h_all, c_all, n, hs, block):
    i32 = jnp.int32
    hh = hs // 2

    def body(h_ref, c_ref, o_ref):
        def packw(a):
            rb = a.astype(jnp.bfloat16).astype(jnp.float32)
            bits = jax.lax.bitcast_convert_type(rb, i32)
            p = jax.lax.shift_right_logical(bits, 16)
            return jax.lax.shift_left(p[:, hh:], 16) | p[:, :hh]

        o_ref[:, :hh] = packw(h_ref[...])
        o_ref[:, hh:] = packw(c_ref[...])

    row = lambda i: (i, 0)
    return pl.pallas_call(
        body,
        grid=(pl.cdiv(n, block),),
        in_specs=[pl.BlockSpec((block, hs), row)] * 2,
        out_specs=pl.BlockSpec((block, hs), row),
        out_shape=jax.ShapeDtypeStruct((n, hs), i32),
        compiler_params=pltpu.CompilerParams(
            dimension_semantics=("arbitrary",)),
    )(h_all, c_all)


def _tc_fused(x, gathered, w_iou, ui0, ui1, uf0, uf1, ufb, biou,
              prev_out, off_blocks, slice_blocks, n, hs, block):
    grid = (slice_blocks,)
    packed = len(gathered) == 2
    f32 = jnp.float32

    def gates(x_ref, w_refs, out_ref, h0b, h1b, c0v, c1v):
        wiou_ref, ui0_ref, ui1_ref, uf0_ref, uf1_ref, ufb_ref, biou_ref = \
            w_refs
        fpre = (jnp.dot(h0b, uf0_ref[...], preferred_element_type=f32)
                + jnp.dot(h1b, uf1_ref[...], preferred_element_type=f32)
                + ufb_ref[...])
        f0 = jax.nn.sigmoid(fpre[:, :hs])
        f1 = jax.nn.sigmoid(fpre[:, hs:])
        cred = f0 * c0v + f1 * c1v
        iou = (jnp.dot(x_ref[...], wiou_ref[...], preferred_element_type=f32)
               + jnp.dot(h0b, ui0_ref[...], preferred_element_type=f32)
               + jnp.dot(h1b, ui1_ref[...], preferred_element_type=f32)
               + biou_ref[...])
        i = jax.nn.sigmoid(iou[:, :hs])
        o = jax.nn.sigmoid(iou[:, hs:2 * hs])
        u = jnp.tanh(iou[:, 2 * hs:])
        c = i * u + cred
        h = o * jnp.tanh(c)
        out_ref[:, :hs] = h
        out_ref[:, hs:] = c

    if packed:
        hh = hs // 2

        def unpackf(w):
            lo = jax.lax.bitcast_convert_type(jax.lax.shift_left(w, 16), f32)
            hi = jax.lax.bitcast_convert_type(w & jnp.int32(-65536), f32)
            return jnp.concatenate([lo, hi], axis=1)

        def body(x_ref, hc0_ref, hc1_ref, *rest):
            hc0b = hc0_ref[...]
            hc1b = hc1_ref[...]
            bf = jnp.bfloat16
            gates(x_ref, rest[:7], rest[-1],
                  unpackf(hc0b[:, :hh]).astype(bf),
                  unpackf(hc1b[:, :hh]).astype(bf),
                  unpackf(hc0b[:, hh:]), unpackf(hc1b[:, hh:]))
        g_specs = [pl.BlockSpec((block, hs), lambda i: (i, 0))] * 2
    else:
        def body(x_ref, h0_ref, h1_ref, c0_ref, c1_ref, *rest):
            gates(x_ref, rest[:7], rest[-1],
                  h0_ref[...], h1_ref[...],
                  c0_ref[...].astype(f32), c1_ref[...].astype(f32))
        g_specs = [pl.BlockSpec((block, hs), lambda i: (i, 0))] * 4

    row_g = lambda i: (i + off_blocks, 0)
    full = lambda i: (0, 0)
    in_specs = (
        [pl.BlockSpec((block, x.shape[1]), row_g)]
        + g_specs
        + [pl.BlockSpec(w.shape, full)
           for w in (w_iou, ui0, ui1, uf0, uf1, ufb, biou)]
    )
    args = [x, *gathered, w_iou, ui0, ui1, uf0, uf1, ufb, biou]
    aliases = {}
    if prev_out is not None:
        in_specs.append(pl.BlockSpec(memory_space=pl.ANY))
        aliases = {len(args): 0}
        args.append(prev_out)
    return pl.pallas_call(
        body,
        grid=grid,
        in_specs=in_specs,
        out_specs=pl.BlockSpec((block, 2 * hs), row_g),
        out_shape=jax.ShapeDtypeStruct((n, 2 * hs), jnp.float32),
        input_output_aliases=aliases,
        compiler_params=pltpu.CompilerParams(
            dimension_semantics=("arbitrary",)),
    )(*args)


_BLOCK = 2048
_KS = 8


def kernel(x, h_all, c_all, child_idx, W_iou, U_iou, U_f_w, U_f_b, b_iou):
    n, _ = x.shape
    hs = h_all.shape[1]
    bf = jnp.bfloat16
    f32 = jnp.float32

    k = pl.cdiv(n, _NW * _CHUNK)
    k += k & 1
    n_pad = _NW * k * _CHUNK

    idx32 = child_idx.astype(jnp.int32)
    idx_t = jnp.pad(idx32.T, ((0, 0), (0, n_pad - n)))

    ks_list = []
    rem = k
    while rem > 0:
        ks_list.append(min(_KS, rem))
        rem -= ks_list[-1]

    ui0, ui1 = U_iou[:hs], U_iou[hs:]
    uf0, uf1 = U_f_w[:hs], U_f_w[hs:]
    ufb = U_f_b.reshape(1, 2 * hs)

    hc16 = _tc_pack(h_all, c_all, n, hs, _BLOCK)
    ui0_16, ui1_16 = ui0.astype(bf), ui1.astype(bf)
    uf0_16, uf1_16 = uf0.astype(bf), uf1.astype(bf)

    out = None
    base = 0
    for s, ks in enumerate(ks_list):
        rows = _NW * ks * _CHUNK
        idx_r = idx_t[:, base:base + rows].reshape(2, _NW, ks, _CHUNK)
        if s == 0:
            gathered = _sc_gather([h_all, c_all], idx_r, rows, ks, hs)
            ws = (ui0, ui1, uf0, uf1)
        else:
            gathered = _sc_gather([hc16], idx_r, rows, ks, hs)
            ws = (ui0_16, ui1_16, uf0_16, uf1_16)
        out = _tc_fused(x, gathered, W_iou, ws[0], ws[1], ws[2], ws[3],
                        ufb, b_iou, out, base // _BLOCK, rows // _BLOCK,
                        n, hs, _BLOCK)
        base += rows
    return out


# speedup vs baseline: 5.4272x; 1.0028x over previous
import functools

import jax
import jax.numpy as jnp
from jax import lax
from jax.experimental import pallas as pl
from jax.experimental.pallas import tpu as pltpu
from jax.experimental.pallas import tpu_sc as plsc

_NC = 2
_NS = 16
_NW = _NC * _NS
_CHUNK = 112


def _sc_gather(tables, idx_r, n_pad, k, w):
    mesh = plsc.VectorSubcoreMesh(
        core_axis_name="c", subcore_axis_name="s",
        num_cores=_NC, num_subcores=_NS)
    dt = tables[0].dtype
    nt = len(tables)
    ns = 2 * nt
    out_sds = jax.ShapeDtypeStruct((n_pad, w), dt)

    @functools.partial(
        pl.kernel,
        mesh=mesh,
        out_type=[out_sds] * ns,
        scratch_types=(
            [pltpu.VMEM((k, _CHUNK), jnp.int32)] * 2
            + [pltpu.VMEM((_CHUNK, w), dt)] * (2 * ns)
            + [pltpu.SemaphoreType.DMA] * 4
        ),
    )
    def gather_kernel(*refs):
        tabs_in = refs[:nt]
        idx_hbm = refs[nt]
        outs = refs[nt + 1:nt + 1 + ns]
        i0_v, i1_v = refs[nt + 1 + ns:nt + 3 + ns]
        bufs = refs[nt + 3 + ns:nt + 3 + ns + 2 * ns]
        bufs_a, bufs_b = bufs[:ns], bufs[ns:]
        gsa, gsb, ssa, ssb = refs[-4:]

        wid = lax.axis_index("s") * _NC + lax.axis_index("c")
        pltpu.sync_copy(idx_hbm.at[0, wid], i0_v)
        pltpu.sync_copy(idx_hbm.at[1, wid], i1_v)
        tabs = tuple(t for t in tabs_in for _ in range(2))
        idxs = (i0_v, i1_v) * nt
        base0 = wid * (k * _CHUNK)

        def gissue(j, bufs, sem):
            for t in range(ns):
                pltpu.async_copy(tabs[t].at[idxs[t].at[j]], bufs[t], sem)

        def gwait(bufs, sem):
            for t in range(ns):
                pltpu.make_async_copy(
                    tabs[t].at[idxs[t].at[0]], bufs[t], sem).wait()

        def sissue(j, bufs, sem):
            for t in range(ns):
                pltpu.async_copy(
                    bufs[t], outs[t].at[pl.ds(base0 + j * _CHUNK, _CHUNK)],
                    sem)

        def sdrain(bufs, sem):
            for t in range(ns):
                pltpu.make_async_copy(
                    bufs[t], outs[t].at[pl.ds(0, _CHUNK)], sem).wait()

        gissue(0, bufs_a, gsa)

        def body(it, carry):
            ja = 2 * it
            jb = ja + 1
            jn = lax.min(ja + 2, k - 1)
            gwait(bufs_a, gsa)

            @pl.when(it > 0)
            def _():
                sdrain(bufs_b, ssb)

            gissue(jb, bufs_b, gsb)
            sissue(ja, bufs_a, ssa)
            gwait(bufs_b, gsb)
            sdrain(bufs_a, ssa)
            gissue(jn, bufs_a, gsa)
            sissue(jb, bufs_b, ssb)
            return carry

        lax.fori_loop(0, k // 2, body, 0)
        gwait(bufs_a, gsa)
        sdrain(bufs_b, ssb)

    return gather_kernel(*tables, idx_r)


def _tc_pack(h_all, c_all, n, hs, block):
    i32 = jnp.int32
    hh = hs // 2

    def body(h_ref, c_ref, o_ref):
        def packw(a):
            rb = a.astype(jnp.bfloat16).astype(jnp.float32)
            bits = jax.lax.bitcast_convert_type(rb, i32)
            p = jax.lax.shift_right_logical(bits, 16)
            return jax.lax.shift_left(p[:, hh:], 16) | p[:, :hh]

        o_ref[:, :hh] = packw(h_ref[...])
        o_ref[:, hh:] = packw(c_ref[...])

    row = lambda i: (i, 0)
    return pl.pallas_call(
        body,
        grid=(pl.cdiv(n, block),),
        in_specs=[pl.BlockSpec((block, hs), row)] * 2,
        out_specs=pl.BlockSpec((block, hs), row),
        out_shape=jax.ShapeDtypeStruct((n, hs), i32),
        compiler_params=pltpu.CompilerParams(
            dimension_semantics=("parallel",)),
    )(h_all, c_all)


def _tc_fused(x, gathered, w_iou, ui0, ui1, uf0, uf1, ufb, biou,
              prev_out, off_blocks, slice_blocks, n, hs, block):
    grid = (slice_blocks,)
    packed = len(gathered) == 2
    f32 = jnp.float32

    def gates(x_ref, w_refs, out_ref, h0b, h1b, c0v, c1v):
        wiou_ref, ui0_ref, ui1_ref, uf0_ref, uf1_ref, ufb_ref, biou_ref = \
            w_refs
        fpre = (jnp.dot(h0b, uf0_ref[...], preferred_element_type=f32)
                + jnp.dot(h1b, uf1_ref[...], preferred_element_type=f32)
                + ufb_ref[...])
        f0 = jax.nn.sigmoid(fpre[:, :hs])
        f1 = jax.nn.sigmoid(fpre[:, hs:])
        cred = f0 * c0v + f1 * c1v
        iou = (jnp.dot(x_ref[...], wiou_ref[...], preferred_element_type=f32)
               + jnp.dot(h0b, ui0_ref[...], preferred_element_type=f32)
               + jnp.dot(h1b, ui1_ref[...], preferred_element_type=f32)
               + biou_ref[...])
        i = jax.nn.sigmoid(iou[:, :hs])
        o = jax.nn.sigmoid(iou[:, hs:2 * hs])
        u = jnp.tanh(iou[:, 2 * hs:])
        c = i * u + cred
        h = o * jnp.tanh(c)
        out_ref[:, :hs] = h
        out_ref[:, hs:] = c

    if packed:
        hh = hs // 2

        def unpackf(w):
            lo = jax.lax.bitcast_convert_type(jax.lax.shift_left(w, 16), f32)
            hi = jax.lax.bitcast_convert_type(w & jnp.int32(-65536), f32)
            return jnp.concatenate([lo, hi], axis=1)

        def body(x_ref, hc0_ref, hc1_ref, *rest):
            hc0b = hc0_ref[...]
            hc1b = hc1_ref[...]
            bf = jnp.bfloat16
            gates(x_ref, rest[:7], rest[-1],
                  unpackf(hc0b[:, :hh]).astype(bf),
                  unpackf(hc1b[:, :hh]).astype(bf),
                  unpackf(hc0b[:, hh:]), unpackf(hc1b[:, hh:]))
        g_specs = [pl.BlockSpec((block, hs), lambda i: (i, 0))] * 2
    else:
        def body(x_ref, h0_ref, h1_ref, c0_ref, c1_ref, *rest):
            gates(x_ref, rest[:7], rest[-1],
                  h0_ref[...], h1_ref[...],
                  c0_ref[...].astype(f32), c1_ref[...].astype(f32))
        g_specs = [pl.BlockSpec((block, hs), lambda i: (i, 0))] * 4

    row_g = lambda i: (i + off_blocks, 0)
    full = lambda i: (0, 0)
    in_specs = (
        [pl.BlockSpec((block, x.shape[1]), row_g)]
        + g_specs
        + [pl.BlockSpec(w.shape, full)
           for w in (w_iou, ui0, ui1, uf0, uf1, ufb, biou)]
    )
    args = [x, *gathered, w_iou, ui0, ui1, uf0, uf1, ufb, biou]
    aliases = {}
    if prev_out is not None:
        in_specs.append(pl.BlockSpec(memory_space=pl.ANY))
        aliases = {len(args): 0}
        args.append(prev_out)
    return pl.pallas_call(
        body,
        grid=grid,
        in_specs=in_specs,
        out_specs=pl.BlockSpec((block, 2 * hs), row_g),
        out_shape=jax.ShapeDtypeStruct((n, 2 * hs), jnp.float32),
        input_output_aliases=aliases,
        compiler_params=pltpu.CompilerParams(
            dimension_semantics=("parallel",)),
    )(*args)


_BLOCK = 2048
_KS = 8


def kernel(x, h_all, c_all, child_idx, W_iou, U_iou, U_f_w, U_f_b, b_iou):
    n, _ = x.shape
    hs = h_all.shape[1]
    bf = jnp.bfloat16
    f32 = jnp.float32

    k = pl.cdiv(n, _NW * _CHUNK)
    k += k & 1
    n_pad = _NW * k * _CHUNK

    idx32 = child_idx.astype(jnp.int32)
    idx_t = jnp.pad(idx32.T, ((0, 0), (0, n_pad - n)))

    ks_list = []
    rem = k
    while rem > 0:
        ks_list.append(min(_KS, rem))
        rem -= ks_list[-1]

    ui0, ui1 = U_iou[:hs], U_iou[hs:]
    uf0, uf1 = U_f_w[:hs], U_f_w[hs:]
    ufb = U_f_b.reshape(1, 2 * hs)

    hc16 = _tc_pack(h_all, c_all, n, hs, _BLOCK)
    ui0_16, ui1_16 = ui0.astype(bf), ui1.astype(bf)
    uf0_16, uf1_16 = uf0.astype(bf), uf1.astype(bf)

    out = None
    base = 0
    for s, ks in enumerate(ks_list):
        rows = _NW * ks * _CHUNK
        idx_r = idx_t[:, base:base + rows].reshape(2, _NW, ks, _CHUNK)
        if s == 0:
            gathered = _sc_gather([h_all, c_all], idx_r, rows, ks, hs)
            ws = (ui0, ui1, uf0, uf1)
        else:
            gathered = _sc_gather([hc16], idx_r, rows, ks, hs)
            ws = (ui0_16, ui1_16, uf0_16, uf1_16)
        out = _tc_fused(x, gathered, W_iou, ws[0], ws[1], ws[2], ws[3],
                        ufb, b_iou, out, base // _BLOCK, rows // _BLOCK,
                        n, hs, _BLOCK)
        base += rows
    return out
